# Initial kernel scaffold; baseline (speedup 1.0000x reference)
import jax
import jax.numpy as jnp
from jax.experimental import pallas as pl


def kernel(rs, rn, params):
    raise NotImplementedError("write your pallas kernel here")


# speedup vs baseline: 22.3406x; 22.3406x over previous
import numpy as np
import jax
import jax.numpy as jnp
from jax.experimental import pallas as pl
from jax.experimental.pallas import tpu as pltpu

_B = 512
_NE = 18
_NN = 4
_EB = 128
_NRBF = 20
_CUT = 5.0
_L = 3
_NBF = 8
_W = 8
_F32 = jnp.float32


def _silu(x):
    return x * jax.nn.sigmoid(x)


def _mm(a, b):
    return jnp.dot(a, b, preferred_element_type=jnp.float32)


def _update(s, v, U, V, w1, b1, w2, b2):
    W = _W
    vf = v.reshape(W * _NE * 3, _EB)
    Uv = _mm(vf, U).reshape(W, _NE, 3, _EB)
    Vv = _mm(vf, V).reshape(W, _NE, 3, _EB)
    Vn = jnp.sqrt(jnp.sum(Vv * Vv, axis=2) + 1e-8)
    cat = jnp.concatenate([s, Vn], axis=-1).reshape(W * _NE, 2 * _EB)
    a = _mm(_silu(_mm(cat, w1) + b1), w2) + b2
    a = a.reshape(W, _NE, 3 * _EB)
    s = s + a[..., :_EB] + a[..., _EB:2 * _EB] * jnp.sum(Uv * Vv, axis=2)
    v = v + a[:, :, None, 2 * _EB:] * Uv
    return s, v


def _painn_kernel(
    rs_ref, rn_ref, X_ref, Y_ref,
    ee_w1, ee_b1, ee_w2, ee_b2, ee_wf, ee_bf,
    ne_w1, ne_b1, ne_w2, ne_b2, ne_wf, ne_bf,
    ue_U, ue_V, ue_w1, ue_b1, ue_w2, ue_b2,
    un_U, un_V, un_w1, un_b1, un_w2, un_b2,
    jw1, jb1, jw2, jb2, bfw,
    jas_ref, bfo_ref,
):
    W = _W
    rs = rs_ref[:]
    rn = rn_ref[:]

    rvec = rs[:, :, None, :] - rs[:, None, :, :]
    d = jnp.sqrt(jnp.sum(rvec * rvec, axis=-1) + 1e-12)
    dc = jnp.maximum(d, 1e-6)
    unit = rvec / dc[..., None]
    nfreq = (jax.lax.broadcasted_iota(jnp.int32, (1, 1, 1, _NRBF), 3)
             .astype(_F32) + 1.0) * (np.pi / _CUT)
    R = jnp.sin(nfreq * dc[..., None]) / dc[..., None]
    R2 = R.reshape(W * _NE * _NE, _NRBF)
    iot_i = jax.lax.broadcasted_iota(jnp.int32, (1, _NE, _NE), 1)
    iot_j = jax.lax.broadcasted_iota(jnp.int32, (1, _NE, _NE), 2)
    mask = (iot_i != iot_j).astype(_F32)
    fc = jnp.where(d < _CUT, 0.5 * (jnp.cos(d * (np.pi / _CUT)) + 1.0), 0.0)
    fc2 = (fc * mask).reshape(W * _NE * _NE, 1)

    rvn = rs[:, None, :, :] - rn[None, :, None, :]
    dn = jnp.sqrt(jnp.sum(rvn * rvn, axis=-1) + 1e-12)
    dnc = jnp.maximum(dn, 1e-6)
    unitn = rvn / dnc[..., None]
    Rn = jnp.sin(nfreq * dnc[..., None]) / dnc[..., None]
    Rn2 = Rn.reshape(W * _NN * _NE, _NRBF)
    fcn = jnp.where(dn < _CUT, 0.5 * (jnp.cos(dn * (np.pi / _CUT)) + 1.0), 0.0)
    fcn2 = fcn.reshape(W * _NN * _NE, 1)

    Yv = Y_ref[:]

    s = jnp.broadcast_to(X_ref[:].reshape(1, 1, _EB), (W, _NE, _EB))
    v = jnp.zeros((W, _NE, 3, _EB), _F32)

    for l in range(_L):
        h = _silu(_mm(s.reshape(W * _NE, _EB), ee_w1[l]) + ee_b1[l])
        phi = _mm(h, ee_w2[l]) + ee_b2[l]
        Wf = (_mm(R2, ee_wf[l]) + ee_bf[l]) * fc2
        x = phi.reshape(W, 1, _NE, 3 * _EB) * Wf.reshape(W, _NE, _NE, 3 * _EB)
        ds = jnp.sum(x[..., :_EB], axis=2)
        dv = jnp.sum(x[:, :, :, _EB:2 * _EB][:, :, :, None, :]
                     * v[:, None, :, :, :], axis=2)
        dv = dv + jnp.sum(x[:, :, :, 2 * _EB:][:, :, :, None, :]
                          * unit[..., None], axis=2)
        s = s + ds
        v = v + dv

        s, v = _update(s, v, ue_U[l], ue_V[l], ue_w1[l], ue_b1[l],
                       ue_w2[l], ue_b2[l])

        hn = _silu(_mm(Yv, ne_w1[l]) + ne_b1[l])
        phin = _mm(hn, ne_w2[l]) + ne_b2[l]
        Wfn = (_mm(Rn2, ne_wf[l]) + ne_bf[l]) * fcn2
        xn = (phin.reshape(1, _NN, 1, 3 * _EB)
              * Wfn.reshape(W, _NN, _NE, 3 * _EB))
        ds = jnp.sum(xn[..., :_EB], axis=1)
        dv = jnp.sum(xn[:, :, :, 2 * _EB:][:, :, :, None, :]
                     * unitn[..., None], axis=1)
        s = s + ds
        v = v + dv

        s, v = _update(s, v, un_U[l], un_V[l], un_w1[l], un_b1[l],
                       un_w2[l], un_b2[l])

    hsum = jnp.sum(s, axis=1)
    jas = _mm(_silu(_mm(hsum, jw1[:]) + jb1[:]), jw2[:]) + jb2[:]
    jas_ref[:] = jas
    bfo_ref[:] = _mm(v.reshape(W * _NE * 3, _EB), bfw[:]).reshape(
        W, _NE * 3, _NBF)


@jax.jit
def kernel(rs, rn, params):
    p = params
    L = _L
    ee, ne = p['msg_ee'], p['msg_ne']
    ue, un = p['upd_e'], p['upd_n']

    ops = [
        rs,
        rn,
        p['X'],
        p['Y'],
        ee['w1'], ee['b1'].reshape(L, 1, _EB),
        ee['w2'], ee['b2'].reshape(L, 1, 3 * _EB),
        ee['wf'], ee['bf'].reshape(L, 1, 3 * _EB),
        ne['w1'], ne['b1'].reshape(L, 1, _EB),
        ne['w2'], ne['b2'].reshape(L, 1, 3 * _EB),
        ne['wf'], ne['bf'].reshape(L, 1, 3 * _EB),
        ue['U'], ue['V'],
        ue['w1'], ue['b1'].reshape(L, 1, _EB),
        ue['w2'], ue['b2'].reshape(L, 1, 3 * _EB),
        un['U'], un['V'],
        un['w1'], un['b1'].reshape(L, 1, _EB),
        un['w2'], un['b2'].reshape(L, 1, 3 * _EB),
        p['jw1'], p['jb1'].reshape(1, _EB),
        p['jw2'], p['jb2'].reshape(1, 1),
        p['bf'],
    ]

    def rep(shape):
        nd = len(shape)
        return pl.BlockSpec(shape, lambda b, _nd=nd: (0,) * _nd)

    in_specs = [pl.BlockSpec((_W, _NE, 3), lambda b: (b, 0, 0))]
    in_specs += [rep(o.shape) for o in ops[1:]]

    jas, bfo = pl.pallas_call(
        _painn_kernel,
        grid=(_B // _W,),
        in_specs=in_specs,
        out_specs=[
            pl.BlockSpec((_W, 1), lambda b: (b, 0)),
            pl.BlockSpec((_W, _NE * 3, _NBF), lambda b: (b, 0, 0)),
        ],
        out_shape=[
            jax.ShapeDtypeStruct((_B, 1), _F32),
            jax.ShapeDtypeStruct((_B, _NE * 3, _NBF), _F32),
        ],
        compiler_params=pltpu.CompilerParams(
            dimension_semantics=("parallel",),
        ),
    )(*ops)

    jastrow = jas[:, 0]
    backflow = bfo.reshape(_B, _NE, 3, _NBF).transpose(0, 3, 1, 2)
    return jastrow, backflow


# speedup vs baseline: 24.1281x; 1.0800x over previous
import numpy as np
import jax
import jax.numpy as jnp
from jax.experimental import pallas as pl
from jax.experimental.pallas import tpu as pltpu

_B = 512
_NE = 18
_NN = 4
_EB = 128
_NRBF = 20
_CUT = 5.0
_L = 3
_NBF = 8
_W = 8
_F32 = jnp.float32


def _silu(x):
    return x * jax.nn.sigmoid(x)


def _mm(a, b):
    return jnp.dot(a, b, preferred_element_type=jnp.float32)


def _update(s, vx, vy, vz, U, V, w1, b1, w2, b2):
    n = _W * _NE
    vcat = jnp.concatenate([vx, vy, vz], axis=0)
    Uv = _mm(vcat, U)
    Vv = _mm(vcat, V)
    Uvx, Uvy, Uvz = Uv[:n], Uv[n:2 * n], Uv[2 * n:]
    Vvx, Vvy, Vvz = Vv[:n], Vv[n:2 * n], Vv[2 * n:]
    Vn = jnp.sqrt(Vvx * Vvx + Vvy * Vvy + Vvz * Vvz + 1e-8)
    cat = jnp.concatenate([s, Vn], axis=-1)
    a = _mm(_silu(_mm(cat, w1) + b1), w2) + b2
    uvdot = Uvx * Vvx + Uvy * Vvy + Uvz * Vvz
    s = s + a[:, :_EB] + a[:, _EB:2 * _EB] * uvdot
    g = a[:, 2 * _EB:]
    return s, vx + g * Uvx, vy + g * Uvy, vz + g * Uvz


def _painn_kernel(
    rs_ref, rn_ref, X_ref, Y_ref,
    ee_w1, ee_b1, ee_w2, ee_b2, ee_wf, ee_bf,
    ne_w1, ne_b1, ne_w2, ne_b2, ne_wf, ne_bf,
    ue_U, ue_V, ue_w1, ue_b1, ue_w2, ue_b2,
    un_U, un_V, un_w1, un_b1, un_w2, un_b2,
    jw1, jb1, jw2, jb2, bfw,
    jas_ref, bx_ref, by_ref, bz_ref,
):
    W = _W
    n = W * _NE
    rs = rs_ref[:]
    rn = rn_ref[:]

    dxx = rs[:, :, None, 0:1] - rs[:, None, :, 0:1]
    dyy = rs[:, :, None, 1:2] - rs[:, None, :, 1:2]
    dzz = rs[:, :, None, 2:3] - rs[:, None, :, 2:3]
    d = jnp.sqrt(dxx * dxx + dyy * dyy + dzz * dzz + 1e-12)
    dc = jnp.maximum(d, 1e-6)
    inv = 1.0 / dc
    shp = (W, _NE, _NE, _EB)
    ux = jnp.broadcast_to(dxx * inv, shp)
    uy = jnp.broadcast_to(dyy * inv, shp)
    uz = jnp.broadcast_to(dzz * inv, shp)
    nfreq = (jax.lax.broadcasted_iota(jnp.int32, (1, 1, 1, _NRBF), 3)
             .astype(_F32) + 1.0) * (np.pi / _CUT)
    R2 = (jnp.sin(nfreq * dc) * inv).reshape(W * _NE * _NE, _NRBF)
    iot_i = jax.lax.broadcasted_iota(jnp.int32, (1, _NE, _NE, 1), 1)
    iot_j = jax.lax.broadcasted_iota(jnp.int32, (1, _NE, _NE, 1), 2)
    mask = (iot_i != iot_j).astype(_F32)
    fc = jnp.where(d < _CUT, 0.5 * (jnp.cos(d * (np.pi / _CUT)) + 1.0), 0.0)
    fc2 = (fc * mask).reshape(W * _NE * _NE, 1)

    nxx = rs[:, None, :, 0:1] - rn[None, :, None, 0:1]
    nyy = rs[:, None, :, 1:2] - rn[None, :, None, 1:2]
    nzz = rs[:, None, :, 2:3] - rn[None, :, None, 2:3]
    dn = jnp.sqrt(nxx * nxx + nyy * nyy + nzz * nzz + 1e-12)
    dnc = jnp.maximum(dn, 1e-6)
    invn = 1.0 / dnc
    shpn = (W, _NN, _NE, _EB)
    unx = jnp.broadcast_to(nxx * invn, shpn)
    uny = jnp.broadcast_to(nyy * invn, shpn)
    unz = jnp.broadcast_to(nzz * invn, shpn)
    Rn2 = (jnp.sin(nfreq * dnc) * invn).reshape(W * _NN * _NE, _NRBF)
    fcn = jnp.where(dn < _CUT, 0.5 * (jnp.cos(dn * (np.pi / _CUT)) + 1.0),
                    0.0)
    fcn2 = fcn.reshape(W * _NN * _NE, 1)

    Yv = Y_ref[:]

    s = jnp.broadcast_to(X_ref[:], (n, _EB))
    vx = jnp.zeros((n, _EB), _F32)
    vy = jnp.zeros((n, _EB), _F32)
    vz = jnp.zeros((n, _EB), _F32)

    for l in range(_L):
        h = _silu(_mm(s, ee_w1[l]) + ee_b1[l])
        phi = (_mm(h, ee_w2[l]) + ee_b2[l]).reshape(W, 1, _NE, 3 * _EB)
        Wf = ((_mm(R2, ee_wf[l]) + ee_bf[l]) * fc2).reshape(
            W, _NE, _NE, 3 * _EB)
        xs = phi[..., :_EB] * Wf[..., :_EB]
        xvv = phi[..., _EB:2 * _EB] * Wf[..., _EB:2 * _EB]
        xvd = phi[..., 2 * _EB:] * Wf[..., 2 * _EB:]
        ds = jnp.sum(xs, axis=2).reshape(n, _EB)
        vxb = vx.reshape(W, 1, _NE, _EB)
        vyb = vy.reshape(W, 1, _NE, _EB)
        vzb = vz.reshape(W, 1, _NE, _EB)
        dvx = jnp.sum(xvv * vxb + xvd * ux, axis=2).reshape(n, _EB)
        dvy = jnp.sum(xvv * vyb + xvd * uy, axis=2).reshape(n, _EB)
        dvz = jnp.sum(xvv * vzb + xvd * uz, axis=2).reshape(n, _EB)
        s = s + ds
        vx = vx + dvx
        vy = vy + dvy
        vz = vz + dvz

        s, vx, vy, vz = _update(s, vx, vy, vz, ue_U[l], ue_V[l], ue_w1[l],
                                ue_b1[l], ue_w2[l], ue_b2[l])

        hn = _silu(_mm(Yv, ne_w1[l]) + ne_b1[l])
        phin = (_mm(hn, ne_w2[l]) + ne_b2[l]).reshape(1, _NN, 1, 3 * _EB)
        Wfn = ((_mm(Rn2, ne_wf[l]) + ne_bf[l]) * fcn2).reshape(
            W, _NN, _NE, 3 * _EB)
        xsn = phin[..., :_EB] * Wfn[..., :_EB]
        xvdn = phin[..., 2 * _EB:] * Wfn[..., 2 * _EB:]
        s = s + jnp.sum(xsn, axis=1).reshape(n, _EB)
        vx = vx + jnp.sum(xvdn * unx, axis=1).reshape(n, _EB)
        vy = vy + jnp.sum(xvdn * uny, axis=1).reshape(n, _EB)
        vz = vz + jnp.sum(xvdn * unz, axis=1).reshape(n, _EB)

        s, vx, vy, vz = _update(s, vx, vy, vz, un_U[l], un_V[l], un_w1[l],
                                un_b1[l], un_w2[l], un_b2[l])

    hsum = jnp.sum(s.reshape(W, _NE, _EB), axis=1)
    jas_ref[:] = _mm(_silu(_mm(hsum, jw1[:]) + jb1[:]), jw2[:]) + jb2[:]
    bx_ref[:] = _mm(vx, bfw[:]).reshape(W, _NE, _NBF)
    by_ref[:] = _mm(vy, bfw[:]).reshape(W, _NE, _NBF)
    bz_ref[:] = _mm(vz, bfw[:]).reshape(W, _NE, _NBF)


@jax.jit
def kernel(rs, rn, params):
    p = params
    L = _L
    ee, ne = p['msg_ee'], p['msg_ne']
    ue, un = p['upd_e'], p['upd_n']

    ops = [
        rs,
        rn,
        p['X'],
        p['Y'],
        ee['w1'], ee['b1'].reshape(L, 1, _EB),
        ee['w2'], ee['b2'].reshape(L, 1, 3 * _EB),
        ee['wf'], ee['bf'].reshape(L, 1, 3 * _EB),
        ne['w1'], ne['b1'].reshape(L, 1, _EB),
        ne['w2'], ne['b2'].reshape(L, 1, 3 * _EB),
        ne['wf'], ne['bf'].reshape(L, 1, 3 * _EB),
        ue['U'], ue['V'],
        ue['w1'], ue['b1'].reshape(L, 1, _EB),
        ue['w2'], ue['b2'].reshape(L, 1, 3 * _EB),
        un['U'], un['V'],
        un['w1'], un['b1'].reshape(L, 1, _EB),
        un['w2'], un['b2'].reshape(L, 1, 3 * _EB),
        p['jw1'], p['jb1'].reshape(1, _EB),
        p['jw2'], p['jb2'].reshape(1, 1),
        p['bf'],
    ]

    def rep(shape):
        nd = len(shape)
        return pl.BlockSpec(shape, lambda b, _nd=nd: (0,) * _nd)

    in_specs = [pl.BlockSpec((_W, _NE, 3), lambda b: (b, 0, 0))]
    in_specs += [rep(o.shape) for o in ops[1:]]

    bfspec = pl.BlockSpec((_W, _NE, _NBF), lambda b: (b, 0, 0))
    bfshape = jax.ShapeDtypeStruct((_B, _NE, _NBF), _F32)
    jas, bx, by, bz = pl.pallas_call(
        _painn_kernel,
        grid=(_B // _W,),
        in_specs=in_specs,
        out_specs=[pl.BlockSpec((_W, 1), lambda b: (b, 0)),
                   bfspec, bfspec, bfspec],
        out_shape=[jax.ShapeDtypeStruct((_B, 1), _F32),
                   bfshape, bfshape, bfshape],
        compiler_params=pltpu.CompilerParams(
            dimension_semantics=("parallel",),
        ),
    )(*ops)

    jastrow = jas[:, 0]
    backflow = jnp.stack([bx, by, bz], axis=-1).transpose(0, 2, 1, 3)
    return jastrow, backflow


# speedup vs baseline: 31.5307x; 1.3068x over previous
import numpy as np
import jax
import jax.numpy as jnp
from jax.experimental import pallas as pl
from jax.experimental.pallas import tpu as pltpu

_B = 512
_NE = 18
_NN = 4
_EB = 128
_NRBF = 20
_NF = _NRBF + 1
_CUT = 5.0
_L = 3
_NBF = 8
_W = 8
_F32 = jnp.float32


def _silu(x):
    return x * jax.nn.sigmoid(x)


def _mm(a, b):
    return jnp.dot(a, b, preferred_element_type=jnp.float32)


def _update(s, vx, vy, vz, U, V, w1, b1, w2, b2):
    n = _W * _NE
    vcat = jnp.concatenate([vx, vy, vz], axis=0)
    Uv = _mm(vcat, U)
    Vv = _mm(vcat, V)
    Uvx, Uvy, Uvz = Uv[:n], Uv[n:2 * n], Uv[2 * n:]
    Vvx, Vvy, Vvz = Vv[:n], Vv[n:2 * n], Vv[2 * n:]
    Vn = jnp.sqrt(Vvx * Vvx + Vvy * Vvy + Vvz * Vvz + 1e-8)
    cat = jnp.concatenate([s, Vn], axis=-1)
    a = _mm(_silu(_mm(cat, w1) + b1), w2) + b2
    uvdot = Uvx * Vvx + Uvy * Vvy + Uvz * Vvz
    s = s + a[:, :_EB] + a[:, _EB:2 * _EB] * uvdot
    g = a[:, 2 * _EB:]
    return s, vx + g * Uvx, vy + g * Uvy, vz + g * Uvz


def _painn_kernel(
    rs_ref, rn_ref, X_ref, Y_ref,
    ee_w1, ee_b1, ee_w2, ee_b2, ee_wfa,
    ne_w1, ne_b1, ne_w2, ne_b2, ne_wfa,
    ue_U, ue_V, ue_w1, ue_b1, ue_w2, ue_b2,
    un_U, un_V, un_w1, un_b1, un_w2, un_b2,
    jw1, jb1, jw2, jb2, bfw,
    jas_ref, bx_ref, by_ref, bz_ref,
    r2a_ref, rn2a_ref, phi_ref, pv_ref,
    ux_ref, uy_ref, uz_ref, unx_ref, uny_ref, unz_ref,
    ds_ref, dvx_ref, dvy_ref, dvz_ref,
):
    W = _W
    n = W * _NE
    rs = rs_ref[:]
    rn = rn_ref[:]

    dxx = rs[:, :, None, 0:1] - rs[:, None, :, 0:1]
    dyy = rs[:, :, None, 1:2] - rs[:, None, :, 1:2]
    dzz = rs[:, :, None, 2:3] - rs[:, None, :, 2:3]
    d = jnp.sqrt(dxx * dxx + dyy * dyy + dzz * dzz + 1e-12)
    dc = jnp.maximum(d, 1e-6)
    inv = 1.0 / dc
    ux_ref[:] = dxx * inv
    uy_ref[:] = dyy * inv
    uz_ref[:] = dzz * inv
    nfreq = (jax.lax.broadcasted_iota(jnp.int32, (1, 1, 1, _NRBF), 3)
             .astype(_F32) + 1.0) * (np.pi / _CUT)
    iot_i = jax.lax.broadcasted_iota(jnp.int32, (1, _NE, _NE, 1), 1)
    iot_j = jax.lax.broadcasted_iota(jnp.int32, (1, _NE, _NE, 1), 2)
    mask = (iot_i != iot_j).astype(_F32)
    fc = jnp.where(d < _CUT, 0.5 * (jnp.cos(d * (np.pi / _CUT)) + 1.0),
                   0.0) * mask
    rbf = jnp.sin(nfreq * dc) * inv * fc
    r2a_ref[:] = jnp.concatenate([rbf, fc], axis=-1).reshape(W, _NE * _NE,
                                                             _NF)

    nxx = rs[:, None, :, 0:1] - rn[None, :, None, 0:1]
    nyy = rs[:, None, :, 1:2] - rn[None, :, None, 1:2]
    nzz = rs[:, None, :, 2:3] - rn[None, :, None, 2:3]
    dn = jnp.sqrt(nxx * nxx + nyy * nyy + nzz * nzz + 1e-12)
    dnc = jnp.maximum(dn, 1e-6)
    invn = 1.0 / dnc
    unx_ref[:] = nxx * invn
    uny_ref[:] = nyy * invn
    unz_ref[:] = nzz * invn
    fcn = jnp.where(dn < _CUT, 0.5 * (jnp.cos(dn * (np.pi / _CUT)) + 1.0),
                    0.0)
    rbfn = jnp.sin(nfreq * dnc) * invn * fcn
    rn2a_ref[:] = jnp.concatenate([rbfn, fcn], axis=-1).reshape(
        W, _NN * _NE, _NF)

    Yv = Y_ref[:]

    s = jnp.broadcast_to(X_ref[:], (n, _EB))
    vx = jnp.zeros((n, _EB), _F32)
    vy = jnp.zeros((n, _EB), _F32)
    vz = jnp.zeros((n, _EB), _F32)

    for l in range(_L):
        h = _silu(_mm(s, ee_w1[l]) + ee_b1[l])
        phi = _mm(h, ee_w2[l]) + ee_b2[l]
        phi_ref[:] = phi.reshape(W, _NE, 3 * _EB)
        pvv = phi[:, _EB:2 * _EB]
        pv_ref[:] = jnp.concatenate(
            [pvv * vx, pvv * vy, pvv * vz], axis=-1).reshape(W, _NE, 3 * _EB)
        wfa = ee_wfa[l]

        def ee_body(b, carry):
            wfb = _mm(r2a_ref[b], wfa).reshape(_NE, _NE, 3 * _EB)
            phib = phi_ref[b]
            pvb = pv_ref[b]
            ds_ref[b] = jnp.sum(phib[None, :, :_EB] * wfb[:, :, :_EB],
                                axis=1)
            t2 = phib[None, :, 2 * _EB:] * wfb[:, :, 2 * _EB:]
            wvv = wfb[:, :, _EB:2 * _EB]
            dvx_ref[b] = jnp.sum(pvb[None, :, :_EB] * wvv
                                 + t2 * ux_ref[b], axis=1)
            dvy_ref[b] = jnp.sum(pvb[None, :, _EB:2 * _EB] * wvv
                                 + t2 * uy_ref[b], axis=1)
            dvz_ref[b] = jnp.sum(pvb[None, :, 2 * _EB:] * wvv
                                 + t2 * uz_ref[b], axis=1)
            return carry

        jax.lax.fori_loop(0, W, ee_body, 0)
        s = s + ds_ref[:].reshape(n, _EB)
        vx = vx + dvx_ref[:].reshape(n, _EB)
        vy = vy + dvy_ref[:].reshape(n, _EB)
        vz = vz + dvz_ref[:].reshape(n, _EB)

        s, vx, vy, vz = _update(s, vx, vy, vz, ue_U[l], ue_V[l], ue_w1[l],
                                ue_b1[l], ue_w2[l], ue_b2[l])

        hn = _silu(_mm(Yv, ne_w1[l]) + ne_b1[l])
        phin = _mm(hn, ne_w2[l]) + ne_b2[l]
        wfna = ne_wfa[l]

        def ne_body(b, carry):
            wfnb = _mm(rn2a_ref[b], wfna).reshape(_NN, _NE, 3 * _EB)
            ds_ref[b] = jnp.sum(phin[:, None, :_EB] * wfnb[:, :, :_EB],
                                axis=0)
            t2 = phin[:, None, 2 * _EB:] * wfnb[:, :, 2 * _EB:]
            dvx_ref[b] = jnp.sum(t2 * unx_ref[b], axis=0)
            dvy_ref[b] = jnp.sum(t2 * uny_ref[b], axis=0)
            dvz_ref[b] = jnp.sum(t2 * unz_ref[b], axis=0)
            return carry

        jax.lax.fori_loop(0, W, ne_body, 0)
        s = s + ds_ref[:].reshape(n, _EB)
        vx = vx + dvx_ref[:].reshape(n, _EB)
        vy = vy + dvy_ref[:].reshape(n, _EB)
        vz = vz + dvz_ref[:].reshape(n, _EB)

        s, vx, vy, vz = _update(s, vx, vy, vz, un_U[l], un_V[l], un_w1[l],
                                un_b1[l], un_w2[l], un_b2[l])

    hsum = jnp.sum(s.reshape(W, _NE, _EB), axis=1)
    jas_ref[:] = _mm(_silu(_mm(hsum, jw1[:]) + jb1[:]), jw2[:]) + jb2[:]
    bx_ref[:] = _mm(vx, bfw[:]).reshape(W, _NE, _NBF)
    by_ref[:] = _mm(vy, bfw[:]).reshape(W, _NE, _NBF)
    bz_ref[:] = _mm(vz, bfw[:]).reshape(W, _NE, _NBF)


@jax.jit
def kernel(rs, rn, params):
    p = params
    L = _L
    ee, ne = p['msg_ee'], p['msg_ne']
    ue, un = p['upd_e'], p['upd_n']

    ee_wfa = jnp.concatenate([ee['wf'], ee['bf'][:, None, :]], axis=1)
    ne_wfa = jnp.concatenate([ne['wf'], ne['bf'][:, None, :]], axis=1)

    ops = [
        rs,
        rn,
        p['X'],
        p['Y'],
        ee['w1'], ee['b1'].reshape(L, 1, _EB),
        ee['w2'], ee['b2'].reshape(L, 1, 3 * _EB),
        ee_wfa,
        ne['w1'], ne['b1'].reshape(L, 1, _EB),
        ne['w2'], ne['b2'].reshape(L, 1, 3 * _EB),
        ne_wfa,
        ue['U'], ue['V'],
        ue['w1'], ue['b1'].reshape(L, 1, _EB),
        ue['w2'], ue['b2'].reshape(L, 1, 3 * _EB),
        un['U'], un['V'],
        un['w1'], un['b1'].reshape(L, 1, _EB),
        un['w2'], un['b2'].reshape(L, 1, 3 * _EB),
        p['jw1'], p['jb1'].reshape(1, _EB),
        p['jw2'], p['jb2'].reshape(1, 1),
        p['bf'],
    ]

    def rep(shape):
        nd = len(shape)
        return pl.BlockSpec(shape, lambda b, _nd=nd: (0,) * _nd)

    in_specs = [pl.BlockSpec((_W, _NE, 3), lambda b: (b, 0, 0))]
    in_specs += [rep(o.shape) for o in ops[1:]]

    scratch = [
        pltpu.VMEM((_W, _NE * _NE, _NF), _F32),
        pltpu.VMEM((_W, _NN * _NE, _NF), _F32),
        pltpu.VMEM((_W, _NE, 3 * _EB), _F32),
        pltpu.VMEM((_W, _NE, 3 * _EB), _F32),
        pltpu.VMEM((_W, _NE, _NE, 1), _F32),
        pltpu.VMEM((_W, _NE, _NE, 1), _F32),
        pltpu.VMEM((_W, _NE, _NE, 1), _F32),
        pltpu.VMEM((_W, _NN, _NE, 1), _F32),
        pltpu.VMEM((_W, _NN, _NE, 1), _F32),
        pltpu.VMEM((_W, _NN, _NE, 1), _F32),
        pltpu.VMEM((_W, _NE, _EB), _F32),
        pltpu.VMEM((_W, _NE, _EB), _F32),
        pltpu.VMEM((_W, _NE, _EB), _F32),
        pltpu.VMEM((_W, _NE, _EB), _F32),
    ]

    bfspec = pl.BlockSpec((_W, _NE, _NBF), lambda b: (b, 0, 0))
    bfshape = jax.ShapeDtypeStruct((_B, _NE, _NBF), _F32)
    jas, bx, by, bz = pl.pallas_call(
        _painn_kernel,
        grid=(_B // _W,),
        in_specs=in_specs,
        out_specs=[pl.BlockSpec((_W, 1), lambda b: (b, 0)),
                   bfspec, bfspec, bfspec],
        out_shape=[jax.ShapeDtypeStruct((_B, 1), _F32),
                   bfshape, bfshape, bfshape],
        scratch_shapes=scratch,
        compiler_params=pltpu.CompilerParams(
            dimension_semantics=("parallel",),
        ),
    )(*ops)

    jastrow = jas[:, 0]
    backflow = jnp.stack([bx, by, bz], axis=-1).transpose(0, 2, 1, 3)
    return jastrow, backflow


# speedup vs baseline: 43.9951x; 1.3953x over previous
import numpy as np
import jax
import jax.numpy as jnp
from jax.experimental import pallas as pl
from jax.experimental.pallas import tpu as pltpu

_B = 512
_NE = 18
_NN = 4
_NP = _NE * _NE
_NPN = _NN * _NE
_EB = 128
_NRBF = 20
_NF = _NRBF + 1
_NFP = 24
_CUT = 5.0
_L = 3
_NBF = 8
_W = 8
_F32 = jnp.float32


def _silu(x):
    return x * jax.nn.sigmoid(x)


def _mm(a, b):
    return jnp.dot(a, b, preferred_element_type=jnp.float32)


def _mmT(a, b):
    return jax.lax.dot_general(a, b, (((0,), (0,)), ((), ())),
                               preferred_element_type=jnp.float32)


def _update(s, vx, vy, vz, U, V, w1, b1, w2, b2):
    n = _W * _NE
    vcat = jnp.concatenate([vx, vy, vz], axis=0)
    Uv = _mm(vcat, U)
    Vv = _mm(vcat, V)
    Uvx, Uvy, Uvz = Uv[:n], Uv[n:2 * n], Uv[2 * n:]
    Vvx, Vvy, Vvz = Vv[:n], Vv[n:2 * n], Vv[2 * n:]
    Vn = jnp.sqrt(Vvx * Vvx + Vvy * Vvy + Vvz * Vvz + 1e-8)
    cat = jnp.concatenate([s, Vn], axis=-1)
    a = _mm(_silu(_mm(cat, w1) + b1), w2) + b2
    uvdot = Uvx * Vvx + Uvy * Vvy + Uvz * Vvz
    s = s + a[:, :_EB] + a[:, _EB:2 * _EB] * uvdot
    g = a[:, 2 * _EB:]
    return s, vx + g * Uvx, vy + g * Uvy, vz + g * Uvz


def _features(dx, dy, dz, mask):
    d = jnp.sqrt(dx * dx + dy * dy + dz * dz + 1e-12)
    dc = jnp.maximum(d, 1e-6)
    inv = 1.0 / dc
    th = d * (np.pi / _CUT)
    s1 = jnp.sin(th)
    c1 = jnp.cos(th)
    fc = jnp.where(d < _CUT, 0.5 * (c1 + 1.0), 0.0)
    if mask is not None:
        fc = fc * mask
    g = inv * fc
    base = []
    sq, cq = s1, c1
    base.append(sq * g)
    for _ in range(_NRBF - 1):
        sq, cq = sq * c1 + cq * s1, cq * c1 - sq * s1
        base.append(sq * g)
    base.append(fc)
    ux = dx * inv
    uy = dy * inv
    uz = dz * inv
    pad = [jnp.zeros_like(fc)] * (_NFP - _NF)
    return (base + pad + [r * ux for r in base] + pad
            + [r * uy for r in base] + pad
            + [r * uz for r in base] + pad)


def _painn_kernel(
    rsl_ref, rnt_ref, X_ref, Y_ref, TE_ref, TJ_ref, TN_ref, EN_ref, mask_ref,
    ee_w1, ee_b1, ee_w2, ee_b2, ee_wfa,
    ne_w1, ne_b1, ne_w2, ne_b2, ne_wfa,
    ue_U, ue_V, ue_w1, ue_b1, ue_w2, ue_b2,
    un_U, un_V, un_w1, un_b1, un_w2, un_b2,
    jw1, jb1, jw2, jb2, bfw,
    jas_ref, bx_ref, by_ref, bz_ref,
    fa_ref, fan_ref, phi_ref, pv_ref,
    ds_ref, dvx_ref, dvy_ref, dvz_ref,
):
    W = _W
    n = W * _NE

    rx = rsl_ref[:, 0, :]
    ry = rsl_ref[:, 1, :]
    rz = rsl_ref[:, 2, :]
    TE = TE_ref[:]
    TJ = TJ_ref[:]
    dx = _mm(rx, TE) - _mm(rx, TJ)
    dy = _mm(ry, TE) - _mm(ry, TJ)
    dz = _mm(rz, TE) - _mm(rz, TJ)
    rows = _features(dx, dy, dz, mask_ref[:])
    for q in range(4 * _NFP):
        fa_ref[:, q, :] = rows[q]

    TN = TN_ref[:]
    EN = EN_ref[:]
    rnt = rnt_ref[:]
    dxn = _mm(rx, TN) - _mm(rnt[0:1, :], EN)
    dyn = _mm(ry, TN) - _mm(rnt[1:2, :], EN)
    dzn = _mm(rz, TN) - _mm(rnt[2:3, :], EN)
    rowsn = _features(dxn, dyn, dzn, None)
    for q in range(4 * _NFP):
        fan_ref[:, q, :] = rowsn[q]

    Yv = Y_ref[:]

    s = jnp.broadcast_to(X_ref[:], (n, _EB))
    vx = jnp.zeros((n, _EB), _F32)
    vy = jnp.zeros((n, _EB), _F32)
    vz = jnp.zeros((n, _EB), _F32)

    for l in range(_L):
        h = _silu(_mm(s, ee_w1[l]) + ee_b1[l])
        phi = _mm(h, ee_w2[l]) + ee_b2[l]
        phi_ref[:] = phi.reshape(W, _NE, 3 * _EB)
        pvv = phi[:, _EB:2 * _EB]
        pv_ref[:] = jnp.concatenate(
            [pvv * vx, pvv * vy, pvv * vz], axis=-1).reshape(W, _NE, 3 * _EB)
        wfa = ee_wfa[l]
        wfvd = wfa[:, 2 * _EB:]

        def ee_body(b, carry):
            A = fa_ref[b]
            wfb = _mmT(A[:_NFP], wfa).reshape(_NE, _NE, 3 * _EB)
            wdx = _mmT(A[_NFP:2 * _NFP], wfvd).reshape(_NE, _NE, _EB)
            wdy = _mmT(A[2 * _NFP:3 * _NFP], wfvd).reshape(_NE, _NE, _EB)
            wdz = _mmT(A[3 * _NFP:], wfvd).reshape(_NE, _NE, _EB)
            phib = phi_ref[b]
            pvb = pv_ref[b]
            pd = phib[None, :, 2 * _EB:]
            ds_ref[b] = jnp.sum(phib[None, :, :_EB] * wfb[:, :, :_EB],
                                axis=1)
            wvv = wfb[:, :, _EB:2 * _EB]
            dvx_ref[b] = jnp.sum(pvb[None, :, :_EB] * wvv + pd * wdx, axis=1)
            dvy_ref[b] = jnp.sum(pvb[None, :, _EB:2 * _EB] * wvv + pd * wdy,
                                 axis=1)
            dvz_ref[b] = jnp.sum(pvb[None, :, 2 * _EB:] * wvv + pd * wdz,
                                 axis=1)
            return carry

        jax.lax.fori_loop(0, W, ee_body, 0)
        s = s + ds_ref[:].reshape(n, _EB)
        vx = vx + dvx_ref[:].reshape(n, _EB)
        vy = vy + dvy_ref[:].reshape(n, _EB)
        vz = vz + dvz_ref[:].reshape(n, _EB)

        s, vx, vy, vz = _update(s, vx, vy, vz, ue_U[l], ue_V[l], ue_w1[l],
                                ue_b1[l], ue_w2[l], ue_b2[l])

        hn = _silu(_mm(Yv, ne_w1[l]) + ne_b1[l])
        phin = _mm(hn, ne_w2[l]) + ne_b2[l]
        wfna = ne_wfa[l]
        wfnvd = wfna[:, 2 * _EB:]

        def ne_body(b, carry):
            A = fan_ref[b]
            wfb = _mmT(A[:_NFP], wfna).reshape(_NN, _NE, 3 * _EB)
            wdx = _mmT(A[_NFP:2 * _NFP], wfnvd).reshape(_NN, _NE, _EB)
            wdy = _mmT(A[2 * _NFP:3 * _NFP], wfnvd).reshape(_NN, _NE, _EB)
            wdz = _mmT(A[3 * _NFP:], wfnvd).reshape(_NN, _NE, _EB)
            pd = phin[:, None, 2 * _EB:]
            ds_ref[b] = jnp.sum(phin[:, None, :_EB] * wfb[:, :, :_EB],
                                axis=0)
            dvx_ref[b] = jnp.sum(pd * wdx, axis=0)
            dvy_ref[b] = jnp.sum(pd * wdy, axis=0)
            dvz_ref[b] = jnp.sum(pd * wdz, axis=0)
            return carry

        jax.lax.fori_loop(0, W, ne_body, 0)
        s = s + ds_ref[:].reshape(n, _EB)
        vx = vx + dvx_ref[:].reshape(n, _EB)
        vy = vy + dvy_ref[:].reshape(n, _EB)
        vz = vz + dvz_ref[:].reshape(n, _EB)

        s, vx, vy, vz = _update(s, vx, vy, vz, un_U[l], un_V[l], un_w1[l],
                                un_b1[l], un_w2[l], un_b2[l])

    hsum = jnp.sum(s.reshape(W, _NE, _EB), axis=1)
    jas_ref[:] = _mm(_silu(_mm(hsum, jw1[:]) + jb1[:]), jw2[:]) + jb2[:]
    bx_ref[:] = _mm(vx, bfw[:]).reshape(W, _NE, _NBF)
    by_ref[:] = _mm(vy, bfw[:]).reshape(W, _NE, _NBF)
    bz_ref[:] = _mm(vz, bfw[:]).reshape(W, _NE, _NBF)


@jax.jit
def kernel(rs, rn, params):
    p = params
    L = _L
    ee, ne = p['msg_ee'], p['msg_ne']
    ue, un = p['upd_e'], p['upd_n']

    zpad = jnp.zeros((L, _NFP - _NF, 3 * _EB), _F32)
    ee_wfa = jnp.concatenate([ee['wf'], ee['bf'][:, None, :], zpad], axis=1)
    ne_wfa = jnp.concatenate([ne['wf'], ne['bf'][:, None, :], zpad], axis=1)

    ii, jj = np.meshgrid(np.arange(_NE), np.arange(_NE), indexing='ij')
    TE = np.zeros((_NE, _NP), np.float32)
    TE[ii.reshape(-1), np.arange(_NP)] = 1.0
    TJ = np.zeros((_NE, _NP), np.float32)
    TJ[jj.reshape(-1), np.arange(_NP)] = 1.0
    mask = (ii != jj).reshape(1, _NP).astype(np.float32)
    aa, ei = np.meshgrid(np.arange(_NN), np.arange(_NE), indexing='ij')
    TN = np.zeros((_NE, _NPN), np.float32)
    TN[ei.reshape(-1), np.arange(_NPN)] = 1.0
    EN = np.zeros((_NN, _NPN), np.float32)
    EN[aa.reshape(-1), np.arange(_NPN)] = 1.0

    ops = [
        jnp.swapaxes(rs, 1, 2),
        rn.T,
        p['X'],
        p['Y'],
        jnp.asarray(TE), jnp.asarray(TJ), jnp.asarray(TN), jnp.asarray(EN),
        jnp.asarray(mask),
        ee['w1'], ee['b1'].reshape(L, 1, _EB),
        ee['w2'], ee['b2'].reshape(L, 1, 3 * _EB),
        ee_wfa,
        ne['w1'], ne['b1'].reshape(L, 1, _EB),
        ne['w2'], ne['b2'].reshape(L, 1, 3 * _EB),
        ne_wfa,
        ue['U'], ue['V'],
        ue['w1'], ue['b1'].reshape(L, 1, _EB),
        ue['w2'], ue['b2'].reshape(L, 1, 3 * _EB),
        un['U'], un['V'],
        un['w1'], un['b1'].reshape(L, 1, _EB),
        un['w2'], un['b2'].reshape(L, 1, 3 * _EB),
        p['jw1'], p['jb1'].reshape(1, _EB),
        p['jw2'], p['jb2'].reshape(1, 1),
        p['bf'],
    ]

    def rep(shape):
        nd = len(shape)
        return pl.BlockSpec(shape, lambda b, _nd=nd: (0,) * _nd)

    in_specs = [pl.BlockSpec((_W, 3, _NE), lambda b: (b, 0, 0))]
    in_specs += [rep(o.shape) for o in ops[1:]]

    scratch = [
        pltpu.VMEM((_W, 4 * _NFP, _NP), _F32),
        pltpu.VMEM((_W, 4 * _NFP, _NPN), _F32),
        pltpu.VMEM((_W, _NE, 3 * _EB), _F32),
        pltpu.VMEM((_W, _NE, 3 * _EB), _F32),
        pltpu.VMEM((_W, _NE, _EB), _F32),
        pltpu.VMEM((_W, _NE, _EB), _F32),
        pltpu.VMEM((_W, _NE, _EB), _F32),
        pltpu.VMEM((_W, _NE, _EB), _F32),
    ]

    bfspec = pl.BlockSpec((_W, _NE, _NBF), lambda b: (b, 0, 0))
    bfshape = jax.ShapeDtypeStruct((_B, _NE, _NBF), _F32)
    jas, bx, by, bz = pl.pallas_call(
        _painn_kernel,
        grid=(_B // _W,),
        in_specs=in_specs,
        out_specs=[pl.BlockSpec((_W, 1), lambda b: (b, 0)),
                   bfspec, bfspec, bfspec],
        out_shape=[jax.ShapeDtypeStruct((_B, 1), _F32),
                   bfshape, bfshape, bfshape],
        scratch_shapes=scratch,
        compiler_params=pltpu.CompilerParams(
            dimension_semantics=("parallel",),
        ),
    )(*ops)

    jastrow = jas[:, 0]
    backflow = jnp.stack([bx, by, bz], axis=-1).transpose(0, 2, 1, 3)
    return jastrow, backflow
